# baseline (speedup 1.0000x reference)
import jax
import jax.numpy as jnp
from jax import lax
from jax.experimental import pallas as pl
from jax.experimental.pallas import tpu as pltpu
from jax.experimental.pallas import tpu_sc as plsc

_NC = 2
_NS = 16
_NW = _NC * _NS
_L = 16

_TBL_ROWS = 115
_TBL_W = 33
_WIDTHS = (7, 16, 4, 6, 26)
_ROFF = (0, 13, 45, 53, 64)
_XROWS = 60


def _sc_gather_body(tbl_hbm, im, id_, iw, is_, ii, yr, xt_hbm,
                    tbl_v, idx_v, yr_v, buf_v, sem):
  wid = lax.axis_index("s") * _NC + lax.axis_index("c")
  rows = yr.shape[0] // _NW
  base = wid * rows

  idxs = (im, id_, iw, is_, ii)

  cps = [pltpu.async_copy(tbl_hbm, tbl_v, sem)]
  for f in range(5):
    cps.append(pltpu.async_copy(
        idxs[f].at[pl.ds(base, rows)], idx_v.at[pl.ds(f * rows, rows)], sem))
  cps.append(pltpu.async_copy(yr.at[pl.ds(base, rows)], yr_v, sem))
  for cp in cps:
    cp.wait()

  @plsc.parallel_loop(0, rows // _L, unroll=4)
  def block(rb):
    off = rb * _L
    g = 0
    for f in range(5):
      iv = idx_v[pl.ds(f * rows + off, _L)] * _TBL_W + _ROFF[f] * _TBL_W
      for c in range(_WIDTHS[f]):
        v = plsc.load_gather(tbl_v, [iv + c])
        buf_v[g, pl.ds(off, _L)] = v
        g += 1
    buf_v[59, pl.ds(off, _L)] = yr_v[pl.ds(off, _L)]

  pltpu.sync_copy(buf_v, xt_hbm.at[:, pl.ds(base, rows)])


def _sc_gather(tbl, idxs, yr, batch):
  rows_per_w = batch // _NW
  kern = pl.kernel(
      _sc_gather_body,
      out_type=jax.ShapeDtypeStruct((_XROWS, batch), jnp.float32),
      mesh=plsc.VectorSubcoreMesh(core_axis_name="c", subcore_axis_name="s"),
      compiler_params=pltpu.CompilerParams(needs_layout_passes=False),
      scratch_types=[
          pltpu.VMEM((_TBL_ROWS * _TBL_W,), jnp.float32),
          pltpu.VMEM((5 * rows_per_w,), jnp.int32),
          pltpu.VMEM((rows_per_w,), jnp.float32),
          pltpu.VMEM((_XROWS, rows_per_w), jnp.float32),
          pltpu.SemaphoreType.DMA,
      ],
  )
  return kern(tbl, *idxs, yr)


def _tc_mlp_body(xt_ref, w1_ref, b1_ref, w2_ref, b2_ref, w3_ref, b3_ref,
                 o_ref):
  def mmT(a, b):
    return lax.dot_general(a, b, (((0,), (0,)), ((), ())),
                           preferred_element_type=jnp.float32,
                           precision=lax.Precision.HIGHEST)

  w1 = w1_ref[...]
  w1r = jnp.concatenate([w1[1:60], w1[0:1]], axis=0)
  h = jnp.maximum(mmT(w1r, xt_ref[...]) + b1_ref[...], 0.0)
  h = jnp.maximum(mmT(w2_ref[...], h) + b2_ref[...], 0.0)
  o = jnp.sum(h * w3_ref[...], axis=0, keepdims=True) + b3_ref[...]
  o_ref[...] = jnp.reshape(o, (o.shape[1], 1))


def _tc_mlp(xt, w1, b1, w2, b2, w3, b3, blk=8192):
  batch = xt.shape[1]
  grid = batch // blk
  full = lambda a: pl.BlockSpec(a.shape, lambda i: (0,) * a.ndim)
  return pl.pallas_call(
      _tc_mlp_body,
      grid=(grid,),
      in_specs=[
          pl.BlockSpec((_XROWS, blk), lambda i: (0, i)),
          full(w1), full(b1), full(w2), full(b2), full(w3), full(b3),
      ],
      out_specs=pl.BlockSpec((blk, 1), lambda i: (i, 0)),
      out_shape=jax.ShapeDtypeStruct((batch, 1), jnp.float32),
  )(xt, w1, b1, w2, b2, w3, b3)


def kernel(year, month, day, weekday, stores, items,
           emb_month, emb_day, emb_weekday, emb_stores, emb_items,
           W1, b1, W2, b2, W3, b3):
  batch = year.shape[0]
  pad = lambda t: jnp.pad(t, ((0, 0), (0, _TBL_W - t.shape[1])))
  tbl = jnp.concatenate(
      [pad(emb_month), pad(emb_day), pad(emb_weekday), pad(emb_stores),
       pad(emb_items)], axis=0).reshape(-1)
  idxs = tuple(a.reshape(batch) for a in (month, day, weekday, stores, items))

  xt = _sc_gather(tbl, idxs, year.reshape(batch), batch)
  return _tc_mlp(xt, W1, b1.reshape(-1, 1), W2, b2.reshape(-1, 1), W3,
                 b3.reshape(-1, 1))


# speedup vs baseline: 1.1928x; 1.1928x over previous
import jax
import jax.numpy as jnp
from jax import lax
from jax.experimental import pallas as pl
from jax.experimental.pallas import tpu as pltpu
from jax.experimental.pallas import tpu_sc as plsc

_NC = 2
_NS = 16
_NW = _NC * _NS
_L = 16

_TBL_ROWS = 115
_TBL_W = 33
_WIDTHS = (7, 16, 4, 6, 26)
_ROFF = (0, 13, 45, 53, 64)
_XROWS = 60


def _sc_gather_body(tbl_hbm, im, id_, iw, is_, ii, yr, xt_hbm,
                    tbl_v, idx_v, yr_v, buf_v, sem):
  wid = lax.axis_index("s") * _NC + lax.axis_index("c")
  rows = yr.shape[0] // _NW
  base = wid * rows

  idxs = (im, id_, iw, is_, ii)

  cps = [pltpu.async_copy(tbl_hbm, tbl_v, sem)]
  for f in range(5):
    cps.append(pltpu.async_copy(
        idxs[f].at[pl.ds(base, rows)], idx_v.at[pl.ds(f * rows, rows)], sem))
  cps.append(pltpu.async_copy(yr.at[pl.ds(base, rows)], yr_v, sem))
  for cp in cps:
    cp.wait()

  @plsc.parallel_loop(0, rows // _L, unroll=4)
  def block(rb):
    off = rb * _L
    g = 0
    for f in range(5):
      iv = idx_v[pl.ds(f * rows + off, _L)] * _TBL_W + _ROFF[f] * _TBL_W
      for c in range(_WIDTHS[f]):
        v = plsc.load_gather(tbl_v, [iv + c])
        buf_v[g, pl.ds(off, _L)] = v
        g += 1
    buf_v[59, pl.ds(off, _L)] = yr_v[pl.ds(off, _L)]

  pltpu.sync_copy(buf_v, xt_hbm.at[:, pl.ds(base, rows)])


def _sc_gather(tbl, idxs, yr, batch):
  rows_per_w = batch // _NW
  kern = pl.kernel(
      _sc_gather_body,
      out_type=jax.ShapeDtypeStruct((_XROWS, batch), jnp.float32),
      mesh=plsc.VectorSubcoreMesh(core_axis_name="c", subcore_axis_name="s"),
      compiler_params=pltpu.CompilerParams(needs_layout_passes=False),
      scratch_types=[
          pltpu.VMEM((_TBL_ROWS * _TBL_W,), jnp.float32),
          pltpu.VMEM((5 * rows_per_w,), jnp.int32),
          pltpu.VMEM((rows_per_w,), jnp.float32),
          pltpu.VMEM((_XROWS, rows_per_w), jnp.float32),
          pltpu.SemaphoreType.DMA,
      ],
  )
  return kern(tbl, *idxs, yr)


def _tc_mlp_body(xt_ref, w1_ref, b1_ref, w2_ref, b2_ref, w3_ref, b3_ref,
                 o_ref):
  def mmT(a, b):
    return lax.dot_general(a, b, (((0,), (0,)), ((), ())),
                           preferred_element_type=jnp.float32,
                           precision=lax.Precision.HIGHEST)

  w1 = w1_ref[...]
  w1r = jnp.concatenate([w1[1:60], w1[0:1]], axis=0)
  h = jnp.maximum(mmT(w1r, xt_ref[...]) + b1_ref[...], 0.0)
  h = jnp.maximum(mmT(w2_ref[...], h) + b2_ref[...], 0.0)
  o_ref[...] = jnp.sum(h * w3_ref[...], axis=0, keepdims=True) + b3_ref[...]


def _tc_mlp(xt, w1, b1, w2, b2, w3, b3, blk=8192):
  batch = xt.shape[1]
  grid = batch // blk
  full = lambda a: pl.BlockSpec(a.shape, lambda i: (0,) * a.ndim)
  return pl.pallas_call(
      _tc_mlp_body,
      grid=(grid,),
      in_specs=[
          pl.BlockSpec((_XROWS, blk), lambda i: (0, i)),
          full(w1), full(b1), full(w2), full(b2), full(w3), full(b3),
      ],
      out_specs=pl.BlockSpec((1, blk), lambda i: (0, i)),
      out_shape=jax.ShapeDtypeStruct((1, batch), jnp.float32),
  )(xt, w1, b1, w2, b2, w3, b3)


def kernel(year, month, day, weekday, stores, items,
           emb_month, emb_day, emb_weekday, emb_stores, emb_items,
           W1, b1, W2, b2, W3, b3):
  batch = year.shape[0]
  pad = lambda t: jnp.pad(t, ((0, 0), (0, _TBL_W - t.shape[1])))
  tbl = jnp.concatenate(
      [pad(emb_month), pad(emb_day), pad(emb_weekday), pad(emb_stores),
       pad(emb_items)], axis=0).reshape(-1)
  idxs = tuple(a.reshape(batch) for a in (month, day, weekday, stores, items))

  xt = _sc_gather(tbl, idxs, year.reshape(batch), batch)
  out_t = _tc_mlp(xt, W1, b1.reshape(-1, 1), W2, b2.reshape(-1, 1), W3,
                  b3.reshape(-1, 1))
  return out_t.reshape(batch, 1)
